# baseline (speedup 1.0000x reference)
import functools

import jax
import jax.numpy as jnp
from jax import lax
from jax.experimental import pallas as pl
from jax.experimental.pallas import tpu as pltpu
from jax.experimental.pallas import tpu_sc as plsc

B = 16384
D = 32
GW = 128
GROUPS = 4


def _sc_gather(user_groups, item_groups, user_gidx, item_gidx):
    info = plsc.get_sparse_core_info()
    nc, ns = info.num_cores, info.num_subcores
    nw = nc * ns
    bpw = B // nw
    nchunks = 2
    cs = bpw // nchunks

    mesh = plsc.VectorSubcoreMesh(core_axis_name="c", subcore_axis_name="s")

    @functools.partial(
        pl.kernel,
        mesh=mesh,
        out_type=[
            jax.ShapeDtypeStruct((B, GW), jnp.float32),
            jax.ShapeDtypeStruct((B, GW), jnp.float32),
        ],
        scratch_types=[
            pltpu.VMEM((bpw,), jnp.int32),
            pltpu.VMEM((bpw,), jnp.int32),
            pltpu.VMEM((cs, GW), jnp.float32),
            pltpu.VMEM((cs, GW), jnp.float32),
            pltpu.SemaphoreType.DMA,
        ],
    )
    def k(ut_hbm, it_hbm, ui_hbm, ii_hbm, uout_hbm, iout_hbm,
          uidx_v, iidx_v, urows_v, irows_v, sem):
        wid = lax.axis_index("s") * nc + lax.axis_index("c")
        base = wid * bpw
        pltpu.sync_copy(ui_hbm.at[pl.ds(base, bpw)], uidx_v)
        pltpu.sync_copy(ii_hbm.at[pl.ds(base, bpw)], iidx_v)

        def body(c):
            off = c * cs
            cu = pltpu.async_copy(
                ut_hbm.at[uidx_v.at[pl.ds(off, cs)]], urows_v, sem)
            ci = pltpu.async_copy(
                it_hbm.at[iidx_v.at[pl.ds(off, cs)]], irows_v, sem)
            cu.wait()
            ci.wait()
            pltpu.sync_copy(urows_v, uout_hbm.at[pl.ds(base + off, cs)])
            pltpu.sync_copy(irows_v, iout_hbm.at[pl.ds(base + off, cs)])

        pl.loop(0, nchunks)(body)

    return k(user_groups, item_groups, user_gidx, item_gidx)


_BLK = 2048


def _mlp_body(u_ref, i_ref, uo_ref, io_ref, d_ref, w1u_ref, w1i_ref,
              w1d_ref, b1_ref, w2_ref, b2_ref, w3_ref, b3_ref, o_ref):
    uo = uo_ref[...]
    io = io_ref[...]
    u = jnp.zeros((_BLK, D), jnp.float32)
    it = jnp.zeros((_BLK, D), jnp.float32)
    for w in range(GROUPS):
        u += jnp.where(uo == w, u_ref[:, w * D:(w + 1) * D], 0.0)
        it += jnp.where(io == w, i_ref[:, w * D:(w + 1) * D], 0.0)
    d = d_ref[...]
    h = jnp.dot(u, w1u_ref[...], preferred_element_type=jnp.float32)
    h += jnp.dot(it, w1i_ref[...], preferred_element_type=jnp.float32)
    h += d * w1d_ref[...]
    h = jnp.maximum(h + b1_ref[...], 0.0)
    h = jnp.dot(h, w2_ref[...], preferred_element_type=jnp.float32)
    h = jnp.maximum(h + b2_ref[...], 0.0)
    o_ref[...] = jnp.dot(h, w3_ref[...], preferred_element_type=jnp.float32) + b3_ref[...]


def _tc_mlp(u_g, i_g, uo, io, dates, W1, b1, W2, b2, W3, b3):
    w1u = W1[:D]
    w1i = W1[D:2 * D]
    w1d = W1[2 * D:2 * D + 1]
    grid = (B // _BLK,)
    row_spec = lambda w: pl.BlockSpec((_BLK, w), lambda i: (i, 0))
    full = lambda a, b: pl.BlockSpec((a, b), lambda i: (0, 0))
    return pl.pallas_call(
        _mlp_body,
        grid=grid,
        in_specs=[
            row_spec(GW),
            row_spec(GW),
            row_spec(1),
            row_spec(1),
            row_spec(1),
            full(D, 64),
            full(D, 64),
            full(1, 64),
            full(1, 64),
            full(64, 32),
            full(1, 32),
            full(32, 2),
            full(1, 2),
        ],
        out_specs=pl.BlockSpec((_BLK, 2), lambda i: (i, 0)),
        out_shape=jax.ShapeDtypeStruct((B, 2), jnp.float32),
    )(u_g, i_g, uo, io, dates, w1u, w1i, w1d, b1.reshape(1, 64),
      W2, b2.reshape(1, 32), W3, b3.reshape(1, 2))


def kernel(user_indices, item_indices, click_dates, user_table, item_table,
           W1, b1, W2, b2, W3, b3):
    ui = user_indices.astype(jnp.int32)
    ii = item_indices.astype(jnp.int32)
    u_groups = user_table.reshape(-1, GW)
    i_groups = item_table.reshape(-1, GW)
    u_g, i_g = _sc_gather(u_groups, i_groups, ui >> 2, ii >> 2)
    uo = (ui & 3).reshape(B, 1)
    io = (ii & 3).reshape(B, 1)
    return _tc_mlp(u_g, i_g, uo, io, click_dates, W1, b1, W2, b2, W3, b3)


# speedup vs baseline: 1.5271x; 1.5271x over previous
import functools

import jax
import jax.numpy as jnp
from jax import lax
from jax.experimental import pallas as pl
from jax.experimental.pallas import tpu as pltpu
from jax.experimental.pallas import tpu_sc as plsc

B = 16384
D = 32


def _sc_gather_one(table, idx):
    info = plsc.get_sparse_core_info()
    nc, ns = info.num_cores, info.num_subcores
    nw = nc * ns
    bpw = B // nw
    ck = 16
    nck = bpw // ck

    mesh = plsc.VectorSubcoreMesh(core_axis_name="c", subcore_axis_name="s")

    @functools.partial(
        pl.kernel,
        mesh=mesh,
        out_type=jax.ShapeDtypeStruct((B, D), jnp.float32),
        scratch_types=[
            pltpu.VMEM((bpw,), jnp.int32),
            pltpu.VMEM((bpw, D), jnp.float32),
            pltpu.SemaphoreType.DMA,
        ],
    )
    def k(t_hbm, i_hbm, out_hbm, idx_v, rows_v, sem):
        wid = lax.axis_index("s") * nc + lax.axis_index("c")
        base = wid * bpw
        pltpu.sync_copy(i_hbm.at[pl.ds(base, bpw)], idx_v)

        def issue(c):
            off = c * ck
            vals = idx_v[pl.ds(off, ck)]
            for j in range(ck):
                pltpu.async_copy(
                    t_hbm.at[pl.ds(vals[j], 1)],
                    rows_v.at[pl.ds(off + j, 1)], sem)

        def drain(c):
            off = c * ck
            pltpu.make_async_copy(
                t_hbm.at[pl.ds(0, ck)], rows_v.at[pl.ds(off, ck)], sem).wait()

        issue(0)

        def step(c):
            issue(c)
            drain(c - 1)

        pl.loop(1, nck)(step)
        drain(nck - 1)

        pltpu.sync_copy(rows_v, out_hbm.at[pl.ds(base, bpw)])

    return k(table, idx)


def _sc_gather(user_table, item_table, user_idx, item_idx):
    return (_sc_gather_one(user_table, user_idx),
            _sc_gather_one(item_table, item_idx))


_BLK = 2048


def _mlp_body(u_ref, i_ref, d_ref, w1u_ref, w1i_ref, w1d_ref, b1_ref,
              w2_ref, b2_ref, w3_ref, b3_ref, o_ref):
    u = u_ref[...]
    it = i_ref[...]
    d = d_ref[...]
    h = jnp.dot(u, w1u_ref[...], preferred_element_type=jnp.float32)
    h += jnp.dot(it, w1i_ref[...], preferred_element_type=jnp.float32)
    h += d * w1d_ref[...]
    h = jnp.maximum(h + b1_ref[...], 0.0)
    h = jnp.dot(h, w2_ref[...], preferred_element_type=jnp.float32)
    h = jnp.maximum(h + b2_ref[...], 0.0)
    o_ref[...] = jnp.dot(h, w3_ref[...], preferred_element_type=jnp.float32) + b3_ref[...]


def _tc_mlp(u_emb, i_emb, dates, W1, b1, W2, b2, W3, b3):
    w1u = W1[:D]
    w1i = W1[D:2 * D]
    w1d = W1[2 * D:2 * D + 1]
    grid = (B // _BLK,)
    row_spec = lambda w: pl.BlockSpec((_BLK, w), lambda i: (i, 0))
    full = lambda a, b: pl.BlockSpec((a, b), lambda i: (0, 0))
    return pl.pallas_call(
        _mlp_body,
        grid=grid,
        in_specs=[
            row_spec(D),
            row_spec(D),
            row_spec(1),
            full(D, 64),
            full(D, 64),
            full(1, 64),
            full(1, 64),
            full(64, 32),
            full(1, 32),
            full(32, 2),
            full(1, 2),
        ],
        out_specs=pl.BlockSpec((_BLK, 2), lambda i: (i, 0)),
        out_shape=jax.ShapeDtypeStruct((B, 2), jnp.float32),
    )(u_emb, i_emb, dates, w1u, w1i, w1d, b1.reshape(1, 64),
      W2, b2.reshape(1, 32), W3, b3.reshape(1, 2))


def kernel(user_indices, item_indices, click_dates, user_table, item_table,
           W1, b1, W2, b2, W3, b3):
    ui = user_indices.astype(jnp.int32)
    ii = item_indices.astype(jnp.int32)
    u_emb, i_emb = _sc_gather(user_table, item_table, ui, ii)
    return _tc_mlp(u_emb, i_emb, click_dates, W1, b1, W2, b2, W3, b3)


# speedup vs baseline: 2.7858x; 1.8242x over previous
import functools

import jax
import jax.numpy as jnp
from jax import lax
from jax.experimental import pallas as pl
from jax.experimental.pallas import tpu as pltpu
from jax.experimental.pallas import tpu_sc as plsc

B = 16384
D = 32
N_ROWS = 1000000
GROUPS = 4
_RC = 8192
_RQ = _RC // GROUPS
_NBLK = (N_ROWS + _RC - 1) // _RC
N_G = _NBLK * _RQ


def _repack_body(t_ref, eye_ref, o_ref):
    x = t_ref[...]
    eye = eye_ref[...]
    x4 = jnp.concatenate(
        [x[:, w * _RQ:(w + 1) * _RQ] for w in range(GROUPS)], axis=0)
    dn = (((0,), (0,)), ((), ()))
    o_ref[...] = lax.dot_general(x4, eye, dn,
                                 preferred_element_type=jnp.float32)


def _tc_repack(table_t, eye):
    return pl.pallas_call(
        _repack_body,
        grid=(_NBLK,),
        in_specs=[
            pl.BlockSpec((D, _RC), lambda i: (0, i)),
            pl.BlockSpec((GROUPS * D, GROUPS * D), lambda i: (0, 0)),
        ],
        out_specs=pl.BlockSpec((_RQ, GROUPS * D), lambda i: (i, 0)),
        out_shape=jax.ShapeDtypeStruct((N_G, GROUPS * D), jnp.float32),
    )(table_t, eye)


def _sc_gather(user_groups, item_groups, user_gidx, item_gidx):
    info = plsc.get_sparse_core_info()
    nc, ns = info.num_cores, info.num_subcores
    nw = nc * ns
    bpw = B // nw
    nchunks = 2
    cs = bpw // nchunks

    mesh = plsc.VectorSubcoreMesh(core_axis_name="c", subcore_axis_name="s")

    @functools.partial(
        pl.kernel,
        mesh=mesh,
        out_type=[
            jax.ShapeDtypeStruct((B, GROUPS * D), jnp.float32),
            jax.ShapeDtypeStruct((B, GROUPS * D), jnp.float32),
        ],
        scratch_types=[
            pltpu.VMEM((bpw,), jnp.int32),
            pltpu.VMEM((bpw,), jnp.int32),
            pltpu.VMEM((cs, GROUPS * D), jnp.float32),
            pltpu.VMEM((cs, GROUPS * D), jnp.float32),
            pltpu.SemaphoreType.DMA,
        ],
    )
    def k(ut_hbm, it_hbm, ui_hbm, ii_hbm, uout_hbm, iout_hbm,
          uidx_v, iidx_v, urows_v, irows_v, sem):
        wid = lax.axis_index("s") * nc + lax.axis_index("c")
        base = wid * bpw
        pltpu.sync_copy(ui_hbm.at[pl.ds(base, bpw)], uidx_v)
        pltpu.sync_copy(ii_hbm.at[pl.ds(base, bpw)], iidx_v)

        def body(c):
            off = c * cs
            cu = pltpu.async_copy(
                ut_hbm.at[uidx_v.at[pl.ds(off, cs)]], urows_v, sem)
            ci = pltpu.async_copy(
                it_hbm.at[iidx_v.at[pl.ds(off, cs)]], irows_v, sem)
            cu.wait()
            ci.wait()
            pltpu.sync_copy(urows_v, uout_hbm.at[pl.ds(base + off, cs)])
            pltpu.sync_copy(irows_v, iout_hbm.at[pl.ds(base + off, cs)])

        pl.loop(0, nchunks)(body)

    return k(user_groups, item_groups, user_gidx, item_gidx)


_BLK = 2048


def _mlp_body(u_ref, i_ref, uo_ref, io_ref, d_ref, w1u_ref, w1i_ref,
              w1d_ref, b1_ref, w2_ref, b2_ref, w3_ref, b3_ref, o_ref):
    uo = uo_ref[...]
    io = io_ref[...]
    lane_w = lax.broadcasted_iota(jnp.int32, (_BLK, GROUPS * D), 1) // D
    zu = jnp.where(lane_w == uo, u_ref[...], 0.0)
    zi = jnp.where(lane_w == io, i_ref[...], 0.0)
    d = d_ref[...]
    h = jnp.dot(zu, w1u_ref[...], preferred_element_type=jnp.float32)
    h += jnp.dot(zi, w1i_ref[...], preferred_element_type=jnp.float32)
    h += d * w1d_ref[...]
    h = jnp.maximum(h + b1_ref[...], 0.0)
    h = jnp.dot(h, w2_ref[...], preferred_element_type=jnp.float32)
    h = jnp.maximum(h + b2_ref[...], 0.0)
    o_ref[...] = jnp.dot(h, w3_ref[...], preferred_element_type=jnp.float32) + b3_ref[...]


def _tc_mlp(u_g, i_g, uo, io, dates, W1, b1, W2, b2, W3, b3):
    w1u = jnp.tile(W1[:D], (GROUPS, 1))
    w1i = jnp.tile(W1[D:2 * D], (GROUPS, 1))
    w1d = W1[2 * D:2 * D + 1]
    grid = (B // _BLK,)
    row_spec = lambda w: pl.BlockSpec((_BLK, w), lambda i: (i, 0))
    full = lambda a, b: pl.BlockSpec((a, b), lambda i: (0, 0))
    return pl.pallas_call(
        _mlp_body,
        grid=grid,
        in_specs=[
            row_spec(GROUPS * D),
            row_spec(GROUPS * D),
            row_spec(1),
            row_spec(1),
            row_spec(1),
            full(GROUPS * D, 64),
            full(GROUPS * D, 64),
            full(1, 64),
            full(1, 64),
            full(64, 32),
            full(1, 32),
            full(32, 2),
            full(1, 2),
        ],
        out_specs=pl.BlockSpec((_BLK, 2), lambda i: (i, 0)),
        out_shape=jax.ShapeDtypeStruct((B, 2), jnp.float32),
    )(u_g, i_g, uo, io, dates, w1u, w1i, w1d, b1.reshape(1, 64),
      W2, b2.reshape(1, 32), W3, b3.reshape(1, 2))


def kernel(user_indices, item_indices, click_dates, user_table, item_table,
           W1, b1, W2, b2, W3, b3):
    ui = user_indices.astype(jnp.int32)
    ii = item_indices.astype(jnp.int32)
    eye = jnp.eye(GROUPS * D, dtype=jnp.float32)
    u_packed = _tc_repack(user_table.T, eye)
    i_packed = _tc_repack(item_table.T, eye)
    u_gidx = (ui >> 13) * _RQ + (ui & (_RQ - 1))
    i_gidx = (ii >> 13) * _RQ + (ii & (_RQ - 1))
    u_g, i_g = _sc_gather(u_packed, i_packed, u_gidx, i_gidx)
    uo = ((ui & (_RC - 1)) >> 11).reshape(B, 1)
    io = ((ii & (_RC - 1)) >> 11).reshape(B, 1)
    return _tc_mlp(u_g, i_g, uo, io, click_dates, W1, b1, W2, b2, W3, b3)


# speedup vs baseline: 3.5992x; 1.2920x over previous
import functools

import jax
import jax.numpy as jnp
from jax import lax
from jax.experimental import pallas as pl
from jax.experimental.pallas import tpu as pltpu
from jax.experimental.pallas import tpu_sc as plsc

B = 16384
D = 32
N_ROWS = 1000000
GROUPS = 4
_RC_LOG = 14
_RC = 1 << _RC_LOG
_RQ = _RC // GROUPS
_NBLK = (N_ROWS + _RC - 1) // _RC
N_G = _NBLK * _RQ


def _repack_body(t_ref, eye_ref, o_ref):
    x = t_ref[...]
    eye = eye_ref[...]
    x4 = jnp.concatenate(
        [x[:, w * _RQ:(w + 1) * _RQ] for w in range(GROUPS)], axis=0)
    dn = (((0,), (0,)), ((), ()))
    o_ref[...] = lax.dot_general(x4, eye, dn,
                                 preferred_element_type=jnp.float32)


def _tc_repack(table_t, eye):
    return pl.pallas_call(
        _repack_body,
        grid=(_NBLK,),
        in_specs=[
            pl.BlockSpec((D, _RC), lambda i: (0, i)),
            pl.BlockSpec((GROUPS * D, GROUPS * D), lambda i: (0, 0)),
        ],
        out_specs=pl.BlockSpec((_RQ, GROUPS * D), lambda i: (i, 0)),
        out_shape=jax.ShapeDtypeStruct((N_G, GROUPS * D), jnp.float32),
    )(table_t, eye)


def _sc_gather_one(groups, gidx):
    info = plsc.get_sparse_core_info()
    nc, ns = info.num_cores, info.num_subcores
    nw = nc * ns
    bpw = B // nw
    nchunks = 2
    cs = bpw // nchunks

    mesh = plsc.VectorSubcoreMesh(core_axis_name="c", subcore_axis_name="s")

    @functools.partial(
        pl.kernel,
        mesh=mesh,
        out_type=jax.ShapeDtypeStruct((B, GROUPS * D), jnp.float32),
        scratch_types=[
            pltpu.VMEM((bpw,), jnp.int32),
            pltpu.VMEM((cs, GROUPS * D), jnp.float32),
            pltpu.VMEM((cs, GROUPS * D), jnp.float32),
            pltpu.SemaphoreType.DMA,
        ],
    )
    def k(t_hbm, i_hbm, out_hbm, idx_v, rows_a, rows_b, sem):
        wid = lax.axis_index("s") * nc + lax.axis_index("c")
        base = wid * bpw
        pltpu.sync_copy(i_hbm.at[pl.ds(base, bpw)], idx_v)
        bufs = (rows_a, rows_b)

        def body(c):
            off = c * cs
            buf = bufs[c]
            pltpu.async_copy(
                t_hbm.at[idx_v.at[pl.ds(off, cs)]], buf, sem).wait()
            pltpu.sync_copy(buf, out_hbm.at[pl.ds(base + off, cs)])

        for c in range(nchunks):
            body(c)

    return k(groups, gidx)


_BLK = 2048


def _mlp_body(u_ref, i_ref, uo_ref, io_ref, d_ref, w1u_ref, w1i_ref,
              w1d_ref, b1_ref, w2_ref, b2_ref, w3_ref, b3_ref, o_ref):
    uo = uo_ref[...]
    io = io_ref[...]
    lane_w = lax.broadcasted_iota(jnp.int32, (_BLK, GROUPS * D), 1) // D
    zu = jnp.where(lane_w == uo, u_ref[...], 0.0)
    zi = jnp.where(lane_w == io, i_ref[...], 0.0)
    d = d_ref[...]
    h = jnp.dot(zu, w1u_ref[...], preferred_element_type=jnp.float32)
    h += jnp.dot(zi, w1i_ref[...], preferred_element_type=jnp.float32)
    h += d * w1d_ref[...]
    h = jnp.maximum(h + b1_ref[...], 0.0)
    h = jnp.dot(h, w2_ref[...], preferred_element_type=jnp.float32)
    h = jnp.maximum(h + b2_ref[...], 0.0)
    o_ref[...] = jnp.dot(h, w3_ref[...], preferred_element_type=jnp.float32) + b3_ref[...]


def _tc_mlp(u_g, i_g, uo, io, dates, W1, b1, W2, b2, W3, b3):
    w1u = jnp.tile(W1[:D], (GROUPS, 1))
    w1i = jnp.tile(W1[D:2 * D], (GROUPS, 1))
    w1d = W1[2 * D:2 * D + 1]
    grid = (B // _BLK,)
    row_spec = lambda w: pl.BlockSpec((_BLK, w), lambda i: (i, 0))
    full = lambda a, b: pl.BlockSpec((a, b), lambda i: (0, 0))
    return pl.pallas_call(
        _mlp_body,
        grid=grid,
        in_specs=[
            row_spec(GROUPS * D),
            row_spec(GROUPS * D),
            row_spec(1),
            row_spec(1),
            row_spec(1),
            full(GROUPS * D, 64),
            full(GROUPS * D, 64),
            full(1, 64),
            full(1, 64),
            full(64, 32),
            full(1, 32),
            full(32, 2),
            full(1, 2),
        ],
        out_specs=pl.BlockSpec((_BLK, 2), lambda i: (i, 0)),
        out_shape=jax.ShapeDtypeStruct((B, 2), jnp.float32),
    )(u_g, i_g, uo, io, dates, w1u, w1i, w1d, b1.reshape(1, 64),
      W2, b2.reshape(1, 32), W3, b3.reshape(1, 2))


def kernel(user_indices, item_indices, click_dates, user_table, item_table,
           W1, b1, W2, b2, W3, b3):
    ui = user_indices.astype(jnp.int32)
    ii = item_indices.astype(jnp.int32)
    eye = jnp.eye(GROUPS * D, dtype=jnp.float32)
    u_gidx = (ui >> _RC_LOG) * _RQ + (ui & (_RQ - 1))
    i_gidx = (ii >> _RC_LOG) * _RQ + (ii & (_RQ - 1))
    u_packed = _tc_repack(user_table.T, eye)
    u_g = _sc_gather_one(u_packed, u_gidx)
    i_packed = _tc_repack(item_table.T, eye)
    i_g = _sc_gather_one(i_packed, i_gidx)
    uo = ((ui & (_RC - 1)) >> (_RC_LOG - 2)).reshape(B, 1)
    io = ((ii & (_RC - 1)) >> (_RC_LOG - 2)).reshape(B, 1)
    return _tc_mlp(u_g, i_g, uo, io, click_dates, W1, b1, W2, b2, W3, b3)


# speedup vs baseline: 4.0337x; 1.1207x over previous
import functools

import jax
import jax.numpy as jnp
from jax import lax
from jax.experimental import pallas as pl
from jax.experimental.pallas import tpu as pltpu
from jax.experimental.pallas import tpu_sc as plsc

B = 16384
D = 32
N_ROWS = 1000000
GROUPS = 4
_RC_LOG = 15
_RC = 1 << _RC_LOG
_RQ = _RC // GROUPS
_NBLK = (N_ROWS + _RC - 1) // _RC
N_G = _NBLK * _RQ


def _repack_body(t_ref, eye_ref, o_ref):
    x = t_ref[...]
    eye = eye_ref[...]
    x4 = jnp.concatenate(
        [x[:, w * _RQ:(w + 1) * _RQ] for w in range(GROUPS)], axis=0)
    dn = (((0,), (0,)), ((), ()))
    o_ref[...] = lax.dot_general(x4, eye, dn,
                                 preferred_element_type=jnp.float32)


def _tc_repack(table_t, eye):
    return pl.pallas_call(
        _repack_body,
        grid=(_NBLK,),
        in_specs=[
            pl.BlockSpec((D, _RC), lambda i: (0, i)),
            pl.BlockSpec((GROUPS * D, GROUPS * D), lambda i: (0, 0)),
        ],
        out_specs=pl.BlockSpec((_RQ, GROUPS * D), lambda i: (i, 0)),
        out_shape=jax.ShapeDtypeStruct((N_G, GROUPS * D), jnp.float32),
    )(table_t, eye)


def _sc_gather_one(groups, gidx):
    info = plsc.get_sparse_core_info()
    nc, ns = info.num_cores, info.num_subcores
    nw = nc * ns
    bpw = B // nw
    nchunks = 2
    cs = bpw // nchunks

    mesh = plsc.VectorSubcoreMesh(core_axis_name="c", subcore_axis_name="s")

    @functools.partial(
        pl.kernel,
        mesh=mesh,
        out_type=jax.ShapeDtypeStruct((B, GROUPS * D), jnp.float32),
        scratch_types=[
            pltpu.VMEM((bpw,), jnp.int32),
            pltpu.VMEM((cs, GROUPS * D), jnp.float32),
            pltpu.VMEM((cs, GROUPS * D), jnp.float32),
            pltpu.SemaphoreType.DMA,
        ],
    )
    def k(t_hbm, i_hbm, out_hbm, idx_v, rows_a, rows_b, sem):
        wid = lax.axis_index("s") * nc + lax.axis_index("c")
        base = wid * bpw
        pltpu.sync_copy(i_hbm.at[pl.ds(base, bpw)], idx_v)
        bufs = (rows_a, rows_b)

        def body(c):
            off = c * cs
            buf = bufs[c]
            pltpu.async_copy(
                t_hbm.at[idx_v.at[pl.ds(off, cs)]], buf, sem).wait()
            pltpu.sync_copy(buf, out_hbm.at[pl.ds(base + off, cs)])

        for c in range(nchunks):
            body(c)

    return k(groups, gidx)


_BLK = 2048


def _mlp_body(u_ref, i_ref, uo_ref, io_ref, d_ref, w1u_ref, w1i_ref,
              w1d_ref, b1_ref, w2_ref, b2_ref, w3_ref, b3_ref, o_ref):
    uo = uo_ref[...]
    io = io_ref[...]
    lane_w = lax.broadcasted_iota(jnp.int32, (_BLK, GROUPS * D), 1) // D
    zu = jnp.where(lane_w == uo, u_ref[...], 0.0)
    zi = jnp.where(lane_w == io, i_ref[...], 0.0)
    d = d_ref[...]
    h = jnp.dot(zu, w1u_ref[...], preferred_element_type=jnp.float32)
    h += jnp.dot(zi, w1i_ref[...], preferred_element_type=jnp.float32)
    h += d * w1d_ref[...]
    h = jnp.maximum(h + b1_ref[...], 0.0)
    h = jnp.dot(h, w2_ref[...], preferred_element_type=jnp.float32)
    h = jnp.maximum(h + b2_ref[...], 0.0)
    o_ref[...] = jnp.dot(h, w3_ref[...], preferred_element_type=jnp.float32) + b3_ref[...]


def _tc_mlp(u_g, i_g, uo, io, dates, W1, b1, W2, b2, W3, b3):
    w1u = jnp.tile(W1[:D], (GROUPS, 1))
    w1i = jnp.tile(W1[D:2 * D], (GROUPS, 1))
    w1d = W1[2 * D:2 * D + 1]
    grid = (B // _BLK,)
    row_spec = lambda w: pl.BlockSpec((_BLK, w), lambda i: (i, 0))
    full = lambda a, b: pl.BlockSpec((a, b), lambda i: (0, 0))
    return pl.pallas_call(
        _mlp_body,
        grid=grid,
        in_specs=[
            row_spec(GROUPS * D),
            row_spec(GROUPS * D),
            row_spec(1),
            row_spec(1),
            row_spec(1),
            full(GROUPS * D, 64),
            full(GROUPS * D, 64),
            full(1, 64),
            full(1, 64),
            full(64, 32),
            full(1, 32),
            full(32, 2),
            full(1, 2),
        ],
        out_specs=pl.BlockSpec((_BLK, 2), lambda i: (i, 0)),
        out_shape=jax.ShapeDtypeStruct((B, 2), jnp.float32),
    )(u_g, i_g, uo, io, dates, w1u, w1i, w1d, b1.reshape(1, 64),
      W2, b2.reshape(1, 32), W3, b3.reshape(1, 2))


def kernel(user_indices, item_indices, click_dates, user_table, item_table,
           W1, b1, W2, b2, W3, b3):
    ui = user_indices.astype(jnp.int32)
    ii = item_indices.astype(jnp.int32)
    eye = jnp.eye(GROUPS * D, dtype=jnp.float32)
    u_gidx = (ui >> _RC_LOG) * _RQ + (ui & (_RQ - 1))
    i_gidx = (ii >> _RC_LOG) * _RQ + (ii & (_RQ - 1))
    u_packed = _tc_repack(user_table.T, eye)
    u_g = _sc_gather_one(u_packed, u_gidx)
    i_packed = _tc_repack(item_table.T, eye)
    i_g = _sc_gather_one(i_packed, i_gidx)
    uo = ((ui & (_RC - 1)) >> (_RC_LOG - 2)).reshape(B, 1)
    io = ((ii & (_RC - 1)) >> (_RC_LOG - 2)).reshape(B, 1)
    return _tc_mlp(u_g, i_g, uo, io, click_dates, W1, b1, W2, b2, W3, b3)
